# baseline (speedup 1.0000x reference)
import functools
import math

import jax
import jax.numpy as jnp
import numpy as np
from jax import lax
from jax.experimental import pallas as pl
from jax.experimental.pallas import tpu as pltpu
from jax.experimental.pallas import tpu_sc as plsc

_MAX_LEN = 512


def _make_pe_table(max_len, dim):
    position = np.arange(0, max_len, dtype=np.float64)[:, None]
    div_term = np.exp(
        np.arange(0, dim, 2, dtype=np.float64) * -(math.log(10000.0) / dim))
    pe = np.zeros((max_len, dim), dtype=np.float64)
    pe[:, 0::2] = np.sin(position * div_term)
    pe[:, 1::2] = np.cos(position * div_term)
    return jnp.asarray(pe, dtype=jnp.float32)


_GDN = lax.GatherDimensionNumbers(
    offset_dims=(), collapsed_slice_dims=(0,), start_index_map=(0,))


def _shuffle(v, idx):
    return lax.gather(v, idx[:, None], _GDN, (1,),
                      mode=lax.GatherScatterMode.PROMISE_IN_BOUNDS)


def _xlane_sum(v, lane):
    for sh in (1, 2, 4, 8):
        v = v + _shuffle(v, lane ^ sh)
    return v


def _rsqrt16(x):
    xi = plsc.bitcast(x, jnp.int32)
    yi = jnp.full((16,), 0x5F3759DF, dtype=jnp.int32) - lax.shift_right_logical(
        xi, jnp.full((16,), 1, dtype=jnp.int32))
    y = plsc.bitcast(yi, jnp.float32)
    for _ in range(2):
        y = y * (jnp.float32(1.5) - jnp.float32(0.5) * x * y * y)
    return y


def kernel(x, token_table, ln_gamma, ln_beta):
    B, L = x.shape
    V, D = token_table.shape
    pe = _make_pe_table(_MAX_LEN, D)[:L]

    info = plsc.get_sparse_core_info()
    NC, NS = info.num_cores, info.num_subcores
    NW = NC * NS
    BW = B // NW
    CR = 8
    CT = CR * L
    NCH = BW // CR
    TU = 8
    SEGS = [(0, 80), (80, 80), (160, 40)]
    inv_d = jnp.float32(1.0 / D)
    eps = jnp.float32(1e-5)

    mesh = plsc.VectorSubcoreMesh(core_axis_name="c", subcore_axis_name="s")

    @functools.partial(
        pl.kernel,
        out_type=jax.ShapeDtypeStruct((B, L, D), jnp.float32),
        mesh=mesh,
        compiler_params=pltpu.CompilerParams(
            use_tc_tiling_on_sc=False, needs_layout_passes=False),
        scratch_types=[
            pltpu.VMEM((CR, L), jnp.int32),
            pltpu.VMEM((CR, L, D), jnp.float32),
            pltpu.VMEM((L, D), jnp.float32),
            pltpu.VMEM((D,), jnp.float32),
            pltpu.VMEM((D,), jnp.float32),
            pltpu.SemaphoreType.DMA,
        ],
    )
    def run(x_hbm, tab_hbm, pe_hbm, g_hbm, bt_hbm, out_hbm,
            idx_v, rows_v, pe_v, g_v, bt_v, sem):
        wid = lax.axis_index("s") * NC + lax.axis_index("c")
        pltpu.sync_copy(pe_hbm, pe_v)
        pltpu.sync_copy(g_hbm, g_v)
        pltpu.sync_copy(bt_hbm, bt_v)
        b_base = wid * BW
        lane = lax.iota(jnp.int32, 16)
        gv = [g_v[pl.ds(i * 16, 16)] for i in range(4)]
        bv = [bt_v[pl.ds(i * 16, 16)] for i in range(4)]

        def chunk_body(ci, carry):
            b0 = pl.multiple_of(b_base + ci * CR, CR)
            pltpu.sync_copy(x_hbm.at[pl.ds(b0, CR)], idx_v)
            copies = [
                pltpu.async_copy(tab_hbm.at[idx_v.at[j, pl.ds(o, n)]],
                                 rows_v.at[j, pl.ds(o, n)], sem)
                for j in range(CR) for (o, n) in SEGS
            ]
            for cp in copies:
                cp.wait()

            def t_body(g, c2):
                tb = g * TU
                for k in range(TU):
                    t = tb + k
                    j = t // L
                    m = lax.rem(t, L)
                    r = [rows_v[j, m, pl.ds(i * 16, 16)] for i in range(4)]
                    p = [pe_v[m, pl.ds(i * 16, 16)] for i in range(4)]
                    v = [r[i] + p[i] for i in range(4)]
                    s4 = (v[0] + v[1]) + (v[2] + v[3])
                    q4 = (v[0] * v[0] + v[1] * v[1]) \
                        + (v[2] * v[2] + v[3] * v[3])
                    s = _xlane_sum(s4, lane)
                    q = _xlane_sum(q4, lane)
                    mean = s * inv_d
                    var = q * inv_d - mean * mean
                    inv = _rsqrt16(var + eps)
                    for i in range(4):
                        o = (v[i] - mean) * inv * gv[i] + bv[i]
                        rows_v[j, m, pl.ds(i * 16, 16)] = o
                return c2

            lax.fori_loop(0, CT // TU, t_body, 0)
            pltpu.sync_copy(rows_v, out_hbm.at[pl.ds(b0, CR)])
            return carry

        lax.fori_loop(0, NCH, chunk_body, 0)

    return run(x.astype(jnp.int32), token_table, pe, ln_gamma, ln_beta)


# speedup vs baseline: 1.7991x; 1.7991x over previous
import functools
import math

import jax
import jax.numpy as jnp
import numpy as np
from jax import lax
from jax.experimental import pallas as pl
from jax.experimental.pallas import tpu as pltpu
from jax.experimental.pallas import tpu_sc as plsc

_MAX_LEN = 512


def _make_pe_table(max_len, dim):
    position = np.arange(0, max_len, dtype=np.float64)[:, None]
    div_term = np.exp(
        np.arange(0, dim, 2, dtype=np.float64) * -(math.log(10000.0) / dim))
    pe = np.zeros((max_len, dim), dtype=np.float64)
    pe[:, 0::2] = np.sin(position * div_term)
    pe[:, 1::2] = np.cos(position * div_term)
    return jnp.asarray(pe, dtype=jnp.float32)


_GDN = lax.GatherDimensionNumbers(
    offset_dims=(), collapsed_slice_dims=(0,), start_index_map=(0,))


def _shuffle(v, idx):
    return lax.gather(v, idx[:, None], _GDN, (1,),
                      mode=lax.GatherScatterMode.PROMISE_IN_BOUNDS)


def _xlane_sum(v, lane):
    for sh in (1, 2, 4, 8):
        v = v + _shuffle(v, lane ^ sh)
    return v


def _rsqrt16(x):
    xi = plsc.bitcast(x, jnp.int32)
    yi = jnp.full((16,), 0x5F3759DF, dtype=jnp.int32) - lax.shift_right_logical(
        xi, jnp.full((16,), 1, dtype=jnp.int32))
    y = plsc.bitcast(yi, jnp.float32)
    for _ in range(2):
        y = y * (jnp.float32(1.5) - jnp.float32(0.5) * x * y * y)
    return y


def kernel(x, token_table, ln_gamma, ln_beta):
    B, L = x.shape
    V, D = token_table.shape
    pe = _make_pe_table(_MAX_LEN, D)[:L]

    info = plsc.get_sparse_core_info()
    NC, NS = info.num_cores, info.num_subcores
    NW = NC * NS
    BW = B // NW
    CR = 8
    HR = 4
    HT = HR * L
    NCH = BW // CR
    SEGS = [(0, 80), (80, 80), (160, 40)]
    inv_d = jnp.float32(1.0 / D)
    eps = jnp.float32(1e-5)

    mesh = plsc.VectorSubcoreMesh(core_axis_name="c", subcore_axis_name="s")

    @functools.partial(
        pl.kernel,
        out_type=jax.ShapeDtypeStruct((B, L, D), jnp.float32),
        mesh=mesh,
        compiler_params=pltpu.CompilerParams(
            use_tc_tiling_on_sc=False, needs_layout_passes=False),
        scratch_types=[
            pltpu.VMEM((CR, L), jnp.int32),
            pltpu.VMEM((HR, L, D), jnp.float32),
            pltpu.VMEM((HR, L, D), jnp.float32),
            pltpu.VMEM((L, D), jnp.float32),
            pltpu.VMEM((D,), jnp.float32),
            pltpu.VMEM((D,), jnp.float32),
            pltpu.SemaphoreType.DMA,
        ],
    )
    def run(x_hbm, tab_hbm, pe_hbm, g_hbm, bt_hbm, out_hbm,
            idx_v, rows_v, out_v, pe_v, g_v, bt_v, sem):
        wid = lax.axis_index("s") * NC + lax.axis_index("c")
        pltpu.sync_copy(pe_hbm, pe_v)
        pltpu.sync_copy(g_hbm, g_v)
        pltpu.sync_copy(bt_hbm, bt_v)
        b_base = wid * BW
        lane = lax.iota(jnp.int32, 16)
        gv = [g_v[pl.ds(i * 16, 16)] for i in range(4)]
        bv = [bt_v[pl.ds(i * 16, 16)] for i in range(4)]

        def chunk_body(ci, carry):
            b0 = pl.multiple_of(b_base + ci * CR, CR)
            pltpu.sync_copy(x_hbm.at[pl.ds(b0, CR)], idx_v)
            for h in range(2):
                copies = [
                    pltpu.async_copy(
                        tab_hbm.at[idx_v.at[h * HR + j, pl.ds(o, n)]],
                        rows_v.at[j, pl.ds(o, n)], sem)
                    for j in range(HR) for (o, n) in SEGS
                ]
                for cp in copies:
                    cp.wait()

                @functools.partial(plsc.parallel_loop, 0, HT, unroll=8)
                def t_body(t):
                    j = t // L
                    m = lax.rem(t, L)
                    r = [rows_v[j, m, pl.ds(i * 16, 16)] for i in range(4)]
                    p = [pe_v[m, pl.ds(i * 16, 16)] for i in range(4)]
                    v = [r[i] + p[i] for i in range(4)]
                    s4 = (v[0] + v[1]) + (v[2] + v[3])
                    q4 = (v[0] * v[0] + v[1] * v[1]) \
                        + (v[2] * v[2] + v[3] * v[3])
                    s = _xlane_sum(s4, lane)
                    q = _xlane_sum(q4, lane)
                    mean = s * inv_d
                    var = q * inv_d - mean * mean
                    inv = _rsqrt16(var + eps)
                    for i in range(4):
                        out_v[j, m, pl.ds(i * 16, 16)] = \
                            (v[i] - mean) * inv * gv[i] + bv[i]

                pltpu.sync_copy(out_v, out_hbm.at[pl.ds(b0 + h * HR, HR)])
            return carry

        lax.fori_loop(0, NCH, chunk_body, 0)

    return run(x.astype(jnp.int32), token_table, pe, ln_gamma, ln_beta)
